# baseline (speedup 1.0000x reference)
import functools

import jax
import jax.numpy as jnp
from jax import lax
from jax.experimental import pallas as pl
from jax.experimental.pallas import tpu as pltpu
from jax.experimental.pallas import tpu_sc as plsc

N = 10000
E = 320000
D_IN = 128
HID = 64

NC = 2
NS = 16
NW = NC * NS

CH = 128
CPT = 80
E_PAD = NW * CPT * CH
N_PAD = 10240
RPT = N_PAD // NS
BLK = 512
NBLK = N_PAD // BLK

_mesh = plsc.VectorSubcoreMesh(core_axis_name="c", subcore_axis_name="s")


def _deg_body(dst_ref, out_ref, dstv, ones_v, zb, deg_sh):
    c = lax.axis_index("c")
    s = lax.axis_index("s")
    wid = c * NS + s
    for i in range(8):
        ones_v[pl.ds(i * 16, 16)] = jnp.ones((16,), jnp.float32)
    for i in range(RPT // 16):
        zb[pl.ds(i * 16, 16)] = jnp.zeros((16,), jnp.float32)
    pltpu.sync_copy(zb, deg_sh.at[pl.ds(s * RPT, RPT)])
    pltpu.sync_copy(dst_ref.at[pl.ds(wid * CPT, CPT)], dstv)
    plsc.subcore_barrier()

    def body(j, carry):
        pltpu.sync_copy(ones_v, deg_sh.at[dstv.at[j]], add=True)
        return carry

    lax.fori_loop(0, CPT, body, 0)
    plsc.subcore_barrier()
    pltpu.sync_copy(deg_sh.at[pl.ds(s * RPT, RPT)], zb)
    pltpu.sync_copy(zb, out_ref.at[c, pl.ds(s * RPT, RPT)])


_deg_call = functools.partial(
    pl.kernel,
    out_type=jax.ShapeDtypeStruct((NC, N_PAD), jnp.float32),
    mesh=_mesh,
    scratch_types=[
        pltpu.VMEM((CPT, CH), jnp.int32),
        pltpu.VMEM((CH,), jnp.float32),
        pltpu.VMEM((RPT,), jnp.float32),
        pltpu.VMEM_SHARED((N_PAD,), jnp.float32),
    ],
)(_deg_body)


NBUF = 2


def _agg_body(g_ref, src_ref, dst_ref, out_ref,
              srcv, dstv, rows, bounce, acc_sh, gsems, ssems):
    c = lax.axis_index("c")
    s = lax.axis_index("s")
    wid = c * NS + s

    def zrow(r, carry):
        for cc in range(HID // 16):
            bounce[r, pl.ds(cc * 16, 16)] = jnp.zeros((16,), jnp.float32)
        return carry

    lax.fori_loop(0, RPT, zrow, 0)
    pltpu.sync_copy(bounce, acc_sh.at[pl.ds(s * RPT, RPT)])
    pltpu.sync_copy(src_ref.at[pl.ds(wid * CPT, CPT)], srcv)
    pltpu.sync_copy(dst_ref.at[pl.ds(wid * CPT, CPT)], dstv)
    plsc.subcore_barrier()

    def gather(j, b):
        return pltpu.make_async_copy(g_ref.at[srcv.at[j]], rows.at[b],
                                     gsems.at[b])

    def scatter(j, b):
        return pltpu.async_copy(rows.at[b], acc_sh.at[dstv.at[j]],
                                ssems.at[b], add=True)

    for b in range(NBUF):
        gather(b, b).start()

    def body(i, carry):
        j0 = NBUF * i
        for b in range(NBUF):
            gather(j0 + b, b).wait()
            scatter(j0 + b, b)
        for b in range(NBUF):
            pltpu.make_async_copy(rows.at[b], acc_sh.at[dstv.at[j0 + b]],
                                  ssems.at[b]).wait()

            @pl.when(j0 + b + NBUF < CPT)
            def _():
                gather(j0 + b + NBUF, b).start()

        return carry

    lax.fori_loop(0, CPT // NBUF, body, 0)
    plsc.subcore_barrier()
    pltpu.sync_copy(acc_sh.at[pl.ds(s * RPT, RPT)], bounce)
    pltpu.sync_copy(bounce, out_ref.at[c, pl.ds(s * RPT, RPT)])


_agg_call = functools.partial(
    pl.kernel,
    out_type=jax.ShapeDtypeStruct((NC, N_PAD, HID), jnp.float32),
    mesh=_mesh,
    compiler_params=pltpu.CompilerParams(use_tc_tiling_on_sc=False),
    scratch_types=[
        pltpu.VMEM((CPT, CH), jnp.int32),
        pltpu.VMEM((CPT, CH), jnp.int32),
        pltpu.VMEM((NBUF, CH, HID), jnp.float32),
        pltpu.VMEM((RPT, HID), jnp.float32),
        pltpu.VMEM_SHARED((N_PAD, HID), jnp.float32),
        pltpu.SemaphoreType.DMA((NBUF,)),
        pltpu.SemaphoreType.DMA((NBUF,)),
    ],
)(_agg_body)


def _mm1_body(x_ref, w_ref, d0_ref, d1_ref, g_ref, dinv_ref):
    i = pl.program_id(0)
    deg = 1.0 + d0_ref[...] + d1_ref[...]
    dinv = lax.rsqrt(deg)
    t = jnp.dot(x_ref[...], w_ref[...],
                preferred_element_type=jnp.float32,
                precision=lax.Precision.HIGHEST)
    rows = lax.broadcasted_iota(jnp.int32, (BLK, 1), 0) + i * BLK
    g_ref[...] = jnp.where(rows < N, dinv * t, 0.0)
    dinv_ref[...] = dinv


def _mm1_call(x, w1, d0, d1):
    return pl.pallas_call(
        _mm1_body,
        grid=(NBLK,),
        in_specs=[
            pl.BlockSpec((BLK, D_IN), lambda i: (i, 0)),
            pl.BlockSpec((D_IN, HID), lambda i: (0, 0)),
            pl.BlockSpec((BLK, 1), lambda i: (i, 0)),
            pl.BlockSpec((BLK, 1), lambda i: (i, 0)),
        ],
        out_specs=[
            pl.BlockSpec((BLK, HID), lambda i: (i, 0)),
            pl.BlockSpec((BLK, 1), lambda i: (i, 0)),
        ],
        out_shape=[
            jax.ShapeDtypeStruct((N_PAD, HID), jnp.float32),
            jax.ShapeDtypeStruct((N_PAD, 1), jnp.float32),
        ],
    )(x, w1, d0, d1)


def _mm2_body(agg_ref, g_ref, dinv_ref, b_ref, w_ref, g2_ref):
    i = pl.program_id(0)
    dinv = dinv_ref[...]
    a = agg_ref[0] + agg_ref[1] + g_ref[...]
    h = jnp.maximum(dinv * a + b_ref[...], 0.0)
    t = jnp.dot(h, w_ref[...],
                preferred_element_type=jnp.float32,
                precision=lax.Precision.HIGHEST)
    rows = lax.broadcasted_iota(jnp.int32, (BLK, 1), 0) + i * BLK
    g2_ref[...] = jnp.where(rows < N, dinv * t, 0.0)


def _mm2_call(agg, g, dinv, b1r, w2):
    return pl.pallas_call(
        _mm2_body,
        grid=(NBLK,),
        in_specs=[
            pl.BlockSpec((NC, BLK, HID), lambda i: (0, i, 0)),
            pl.BlockSpec((BLK, HID), lambda i: (i, 0)),
            pl.BlockSpec((BLK, 1), lambda i: (i, 0)),
            pl.BlockSpec((1, HID), lambda i: (0, 0)),
            pl.BlockSpec((HID, HID), lambda i: (0, 0)),
        ],
        out_specs=pl.BlockSpec((BLK, HID), lambda i: (i, 0)),
        out_shape=jax.ShapeDtypeStruct((N_PAD, HID), jnp.float32),
    )(agg, g, dinv, b1r, w2)


def _fin_body(agg_ref, g_ref, dinv_ref, b_ref, wfc_ref, bfc_ref, out_ref, acc):
    i = pl.program_id(0)
    dinv = dinv_ref[...]
    a = agg_ref[0] + agg_ref[1] + g_ref[...]
    h = jnp.maximum(dinv * a + b_ref[...], 0.0)
    rows = lax.broadcasted_iota(jnp.int32, (BLK, 1), 0) + i * BLK
    h = jnp.where(rows < N, h, 0.0)
    part = jnp.sum(h, axis=0, keepdims=True)
    acc[...] = jnp.where(i == 0, part, acc[...] + part)

    @pl.when(i == NBLK - 1)
    def _():
        pooled = acc[...] * (1.0 / N)
        out_ref[...] = jnp.dot(pooled, wfc_ref[...],
                               preferred_element_type=jnp.float32,
                               precision=lax.Precision.HIGHEST) + bfc_ref[...]


def _fin_call(agg, g, dinv, b2r, wfcr, bfcr):
    return pl.pallas_call(
        _fin_body,
        grid=(NBLK,),
        in_specs=[
            pl.BlockSpec((NC, BLK, HID), lambda i: (0, i, 0)),
            pl.BlockSpec((BLK, HID), lambda i: (i, 0)),
            pl.BlockSpec((BLK, 1), lambda i: (i, 0)),
            pl.BlockSpec((1, HID), lambda i: (0, 0)),
            pl.BlockSpec((HID, 2), lambda i: (0, 0)),
            pl.BlockSpec((1, 2), lambda i: (0, 0)),
        ],
        out_specs=pl.BlockSpec((1, 2), lambda i: (0, 0)),
        out_shape=jax.ShapeDtypeStruct((1, 2), jnp.float32),
        scratch_shapes=[pltpu.VMEM((1, HID), jnp.float32)],
    )(agg, g, dinv, b2r, wfcr, bfcr)


def kernel(x, edge_index, W1, b1, W2, b2, Wfc, bfc):
    src = edge_index[0]
    dst = edge_index[1]
    pad = jnp.full((E_PAD - E,), N, dtype=jnp.int32)
    src2d = jnp.concatenate([src, pad]).reshape(NW * CPT, CH)
    dst2d = jnp.concatenate([dst, pad]).reshape(NW * CPT, CH)

    deg2 = _deg_call(dst2d)
    d0 = deg2[0][:, None]
    d1 = deg2[1][:, None]

    g1, dinv = _mm1_call(x, W1, d0, d1)
    agg1 = _agg_call(g1, src2d, dst2d)
    g2 = _mm2_call(agg1, g1, dinv, b1.reshape(1, HID), W2)
    agg2 = _agg_call(g2, src2d, dst2d)
    out = _fin_call(agg2, g2, dinv, b2.reshape(1, HID),
                    Wfc, bfc.reshape(1, 2))
    return out.reshape(2)


# speedup vs baseline: 1.4437x; 1.4437x over previous
import functools

import jax
import jax.numpy as jnp
from jax import lax
from jax.experimental import pallas as pl
from jax.experimental.pallas import tpu as pltpu
from jax.experimental.pallas import tpu_sc as plsc

N = 10000
E = 320000
D_IN = 128
HID = 64

NC = 2
NS = 16
NW = NC * NS

CH = 128
CPT = 80
E_PAD = NW * CPT * CH
N_PAD = 10240
RPT = N_PAD // NS
BLK = 512
NBLK = N_PAD // BLK

_mesh = plsc.VectorSubcoreMesh(core_axis_name="c", subcore_axis_name="s")


def _deg_body(dst_ref, out_ref, dstv, ones_v, zb, deg_sh):
    c = lax.axis_index("c")
    s = lax.axis_index("s")
    wid = c * NS + s
    for i in range(8):
        ones_v[pl.ds(i * 16, 16)] = jnp.ones((16,), jnp.float32)
    for i in range(RPT // 16):
        zb[pl.ds(i * 16, 16)] = jnp.zeros((16,), jnp.float32)
    pltpu.sync_copy(zb, deg_sh.at[pl.ds(s * RPT, RPT)])
    pltpu.sync_copy(dst_ref.at[pl.ds(wid * CPT, CPT)], dstv)
    plsc.subcore_barrier()

    def body(j, carry):
        pltpu.sync_copy(ones_v, deg_sh.at[dstv.at[j]], add=True)
        return carry

    lax.fori_loop(0, CPT, body, 0)
    plsc.subcore_barrier()
    pltpu.sync_copy(deg_sh.at[pl.ds(s * RPT, RPT)], zb)
    pltpu.sync_copy(zb, out_ref.at[c, pl.ds(s * RPT, RPT)])


_deg_call = functools.partial(
    pl.kernel,
    out_type=jax.ShapeDtypeStruct((NC, N_PAD), jnp.float32),
    mesh=_mesh,
    scratch_types=[
        pltpu.VMEM((CPT, CH), jnp.int32),
        pltpu.VMEM((CH,), jnp.float32),
        pltpu.VMEM((RPT,), jnp.float32),
        pltpu.VMEM_SHARED((N_PAD,), jnp.float32),
    ],
)(_deg_body)


def _agg_body(g_ref, src_ref, dst_ref, out_ref,
              srcv, dstv, rows0, rows1, bounce, acc_sh, sem0, sem1):
    c = lax.axis_index("c")
    s = lax.axis_index("s")
    wid = c * NS + s

    def zrow(r, carry):
        for cc in range(HID // 32):
            bounce[r, pl.ds(cc * 32, 32)] = jnp.zeros((32,), jnp.bfloat16)
        return carry

    lax.fori_loop(0, RPT, zrow, 0)
    pltpu.sync_copy(bounce, acc_sh.at[pl.ds(s * RPT, RPT)])
    pltpu.sync_copy(src_ref.at[pl.ds(wid * CPT, CPT)], srcv)
    pltpu.sync_copy(dst_ref.at[pl.ds(wid * CPT, CPT)], dstv)
    plsc.subcore_barrier()

    pltpu.make_async_copy(g_ref.at[srcv.at[0]], rows0, sem0).start()

    def body(i, carry):
        j = 2 * i
        pltpu.make_async_copy(g_ref.at[srcv.at[j + 1]], rows1, sem1).start()
        pltpu.make_async_copy(g_ref.at[srcv.at[j]], rows0, sem0).wait()
        pltpu.sync_copy(rows0, acc_sh.at[dstv.at[j]], add=True)

        @pl.when(j + 2 < CPT)
        def _():
            pltpu.make_async_copy(g_ref.at[srcv.at[j + 2]], rows0, sem0).start()

        pltpu.make_async_copy(g_ref.at[srcv.at[j + 1]], rows1, sem1).wait()
        pltpu.sync_copy(rows1, acc_sh.at[dstv.at[j + 1]], add=True)
        return carry

    lax.fori_loop(0, CPT // 2, body, 0)
    plsc.subcore_barrier()
    pltpu.sync_copy(acc_sh.at[pl.ds(s * RPT, RPT)], bounce)
    pltpu.sync_copy(bounce, out_ref.at[c, pl.ds(s * RPT, RPT)])


_agg_call = functools.partial(
    pl.kernel,
    out_type=jax.ShapeDtypeStruct((NC, N_PAD, HID), jnp.bfloat16),
    mesh=_mesh,
    compiler_params=pltpu.CompilerParams(use_tc_tiling_on_sc=False),
    scratch_types=[
        pltpu.VMEM((CPT, CH), jnp.int32),
        pltpu.VMEM((CPT, CH), jnp.int32),
        pltpu.VMEM((CH, HID), jnp.bfloat16),
        pltpu.VMEM((CH, HID), jnp.bfloat16),
        pltpu.VMEM((RPT, HID), jnp.bfloat16),
        pltpu.VMEM_SHARED((N_PAD, HID), jnp.bfloat16),
        pltpu.SemaphoreType.DMA,
        pltpu.SemaphoreType.DMA,
    ],
)(_agg_body)


def _mm1_body(x_ref, w_ref, d0_ref, d1_ref, g_ref, dinv_ref):
    i = pl.program_id(0)
    deg = 1.0 + d0_ref[...] + d1_ref[...]
    dinv = lax.rsqrt(deg)
    t = jnp.dot(x_ref[...], w_ref[...],
                preferred_element_type=jnp.float32,
                precision=lax.Precision.HIGHEST)
    rows = lax.broadcasted_iota(jnp.int32, (BLK, 1), 0) + i * BLK
    g_ref[...] = jnp.where(rows < N, dinv * t, 0.0).astype(jnp.bfloat16)
    dinv_ref[...] = dinv


def _mm1_call(x, w1, d0, d1):
    return pl.pallas_call(
        _mm1_body,
        grid=(NBLK,),
        in_specs=[
            pl.BlockSpec((BLK, D_IN), lambda i: (i, 0)),
            pl.BlockSpec((D_IN, HID), lambda i: (0, 0)),
            pl.BlockSpec((BLK, 1), lambda i: (i, 0)),
            pl.BlockSpec((BLK, 1), lambda i: (i, 0)),
        ],
        out_specs=[
            pl.BlockSpec((BLK, HID), lambda i: (i, 0)),
            pl.BlockSpec((BLK, 1), lambda i: (i, 0)),
        ],
        out_shape=[
            jax.ShapeDtypeStruct((N_PAD, HID), jnp.bfloat16),
            jax.ShapeDtypeStruct((N_PAD, 1), jnp.float32),
        ],
    )(x, w1, d0, d1)


def _mm2_body(agg_ref, g_ref, dinv_ref, b_ref, w_ref, g2_ref):
    i = pl.program_id(0)
    dinv = dinv_ref[...]
    a = (agg_ref[0].astype(jnp.float32) + agg_ref[1].astype(jnp.float32)
         + g_ref[...].astype(jnp.float32))
    h = jnp.maximum(dinv * a + b_ref[...], 0.0)
    t = jnp.dot(h, w_ref[...],
                preferred_element_type=jnp.float32,
                precision=lax.Precision.HIGHEST)
    rows = lax.broadcasted_iota(jnp.int32, (BLK, 1), 0) + i * BLK
    g2_ref[...] = jnp.where(rows < N, dinv * t, 0.0).astype(jnp.bfloat16)


def _mm2_call(agg, g, dinv, b1r, w2):
    return pl.pallas_call(
        _mm2_body,
        grid=(NBLK,),
        in_specs=[
            pl.BlockSpec((NC, BLK, HID), lambda i: (0, i, 0)),
            pl.BlockSpec((BLK, HID), lambda i: (i, 0)),
            pl.BlockSpec((BLK, 1), lambda i: (i, 0)),
            pl.BlockSpec((1, HID), lambda i: (0, 0)),
            pl.BlockSpec((HID, HID), lambda i: (0, 0)),
        ],
        out_specs=pl.BlockSpec((BLK, HID), lambda i: (i, 0)),
        out_shape=jax.ShapeDtypeStruct((N_PAD, HID), jnp.bfloat16),
    )(agg, g, dinv, b1r, w2)


def _fin_body(agg_ref, g_ref, dinv_ref, b_ref, wfc_ref, bfc_ref, out_ref, acc):
    i = pl.program_id(0)
    dinv = dinv_ref[...]
    a = (agg_ref[0].astype(jnp.float32) + agg_ref[1].astype(jnp.float32)
         + g_ref[...].astype(jnp.float32))
    h = jnp.maximum(dinv * a + b_ref[...], 0.0)
    rows = lax.broadcasted_iota(jnp.int32, (BLK, 1), 0) + i * BLK
    h = jnp.where(rows < N, h, 0.0)
    part = jnp.sum(h, axis=0, keepdims=True)
    acc[...] = jnp.where(i == 0, part, acc[...] + part)

    @pl.when(i == NBLK - 1)
    def _():
        pooled = acc[...] * (1.0 / N)
        out_ref[...] = jnp.dot(pooled, wfc_ref[...],
                               preferred_element_type=jnp.float32,
                               precision=lax.Precision.HIGHEST) + bfc_ref[...]


def _fin_call(agg, g, dinv, b2r, wfcr, bfcr):
    return pl.pallas_call(
        _fin_body,
        grid=(NBLK,),
        in_specs=[
            pl.BlockSpec((NC, BLK, HID), lambda i: (0, i, 0)),
            pl.BlockSpec((BLK, HID), lambda i: (i, 0)),
            pl.BlockSpec((BLK, 1), lambda i: (i, 0)),
            pl.BlockSpec((1, HID), lambda i: (0, 0)),
            pl.BlockSpec((HID, 2), lambda i: (0, 0)),
            pl.BlockSpec((1, 2), lambda i: (0, 0)),
        ],
        out_specs=pl.BlockSpec((1, 2), lambda i: (0, 0)),
        out_shape=jax.ShapeDtypeStruct((1, 2), jnp.float32),
        scratch_shapes=[pltpu.VMEM((1, HID), jnp.float32)],
    )(agg, g, dinv, b2r, wfcr, bfcr)


def kernel(x, edge_index, W1, b1, W2, b2, Wfc, bfc):
    src = edge_index[0]
    dst = edge_index[1]
    pad = jnp.full((E_PAD - E,), N, dtype=jnp.int32)
    src2d = jnp.concatenate([src, pad]).reshape(NW * CPT, CH)
    dst2d = jnp.concatenate([dst, pad]).reshape(NW * CPT, CH)

    deg2 = _deg_call(dst2d)
    d0 = deg2[0][:, None]
    d1 = deg2[1][:, None]

    g1, dinv = _mm1_call(x, W1, d0, d1)
    agg1 = _agg_call(g1, src2d, dst2d)
    g2 = _mm2_call(agg1, g1, dinv, b1.reshape(1, HID), W2)
    agg2 = _agg_call(g2, src2d, dst2d)
    out = _fin_call(agg2, g2, dinv, b2.reshape(1, HID),
                    Wfc, bfc.reshape(1, 2))
    return out.reshape(2)


# speedup vs baseline: 2.5504x; 1.7665x over previous
import functools

import jax
import jax.numpy as jnp
from jax import lax
from jax.experimental import pallas as pl
from jax.experimental.pallas import tpu as pltpu
from jax.experimental.pallas import tpu_sc as plsc

N = 10000
E = 320000
D_IN = 128
HID = 64

NC = 2
NS = 16
NW = NC * NS

CH = 128
CPT = 80
E_PAD = NW * CPT * CH
N_PAD = 10240
RPT = N_PAD // NS
BLK = 512
NBLK = N_PAD // BLK

_mesh = plsc.VectorSubcoreMesh(core_axis_name="c", subcore_axis_name="s")


def _deg_body(dst_ref, out_ref, dstv, ones_v, zb, deg_sh):
    c = lax.axis_index("c")
    s = lax.axis_index("s")
    wid = c * NS + s
    for i in range(8):
        ones_v[pl.ds(i * 16, 16)] = jnp.ones((16,), jnp.float32)
    for i in range(RPT // 16):
        zb[pl.ds(i * 16, 16)] = jnp.zeros((16,), jnp.float32)
    pltpu.sync_copy(zb, deg_sh.at[pl.ds(s * RPT, RPT)])
    pltpu.sync_copy(dst_ref.at[pl.ds(wid * CPT, CPT)], dstv)
    plsc.subcore_barrier()

    def body(j, carry):
        pltpu.sync_copy(ones_v, deg_sh.at[dstv.at[j]], add=True)
        return carry

    lax.fori_loop(0, CPT, body, 0)
    plsc.subcore_barrier()
    pltpu.sync_copy(deg_sh.at[pl.ds(s * RPT, RPT)], zb)
    pltpu.sync_copy(zb, out_ref.at[c, pl.ds(s * RPT, RPT)])


_deg_call = functools.partial(
    pl.kernel,
    out_type=jax.ShapeDtypeStruct((NC, N_PAD), jnp.float32),
    mesh=_mesh,
    scratch_types=[
        pltpu.VMEM((CPT, CH), jnp.int32),
        pltpu.VMEM((CH,), jnp.float32),
        pltpu.VMEM((RPT,), jnp.float32),
        pltpu.VMEM_SHARED((N_PAD,), jnp.float32),
    ],
)(_deg_body)


def _agg_body(g_ref, src_ref, dst_ref, out_ref,
              srcv, dstv, rows0, rows1, bounce, g_sh, acc_sh, sem0, sem1):
    c = lax.axis_index("c")
    s = lax.axis_index("s")
    wid = c * NS + s

    pltpu.sync_copy(g_ref.at[pl.ds(s * RPT, RPT)], bounce)
    pltpu.sync_copy(bounce, g_sh.at[pl.ds(s * RPT, RPT)])

    def zrow(r, carry):
        for cc in range(HID // 32):
            bounce[r, pl.ds(cc * 32, 32)] = jnp.zeros((32,), jnp.bfloat16)
        return carry

    lax.fori_loop(0, RPT, zrow, 0)
    pltpu.sync_copy(bounce, acc_sh.at[pl.ds(s * RPT, RPT)])
    pltpu.sync_copy(src_ref.at[pl.ds(wid * CPT, CPT)], srcv)
    pltpu.sync_copy(dst_ref.at[pl.ds(wid * CPT, CPT)], dstv)
    plsc.subcore_barrier()

    pltpu.make_async_copy(g_sh.at[srcv.at[0]], rows0, sem0).start()

    def body(i, carry):
        j = 2 * i
        pltpu.make_async_copy(g_sh.at[srcv.at[j + 1]], rows1, sem1).start()
        pltpu.make_async_copy(g_sh.at[srcv.at[j]], rows0, sem0).wait()
        pltpu.sync_copy(rows0, acc_sh.at[dstv.at[j]], add=True)

        @pl.when(j + 2 < CPT)
        def _():
            pltpu.make_async_copy(g_sh.at[srcv.at[j + 2]], rows0, sem0).start()

        pltpu.make_async_copy(g_sh.at[srcv.at[j + 1]], rows1, sem1).wait()
        pltpu.sync_copy(rows1, acc_sh.at[dstv.at[j + 1]], add=True)
        return carry

    lax.fori_loop(0, CPT // 2, body, 0)
    plsc.subcore_barrier()
    pltpu.sync_copy(acc_sh.at[pl.ds(s * RPT, RPT)], bounce)
    pltpu.sync_copy(bounce, out_ref.at[c, pl.ds(s * RPT, RPT)])


_agg_call = functools.partial(
    pl.kernel,
    out_type=jax.ShapeDtypeStruct((NC, N_PAD, HID), jnp.bfloat16),
    mesh=_mesh,
    compiler_params=pltpu.CompilerParams(use_tc_tiling_on_sc=False),
    scratch_types=[
        pltpu.VMEM((CPT, CH), jnp.int32),
        pltpu.VMEM((CPT, CH), jnp.int32),
        pltpu.VMEM((CH, HID), jnp.bfloat16),
        pltpu.VMEM((CH, HID), jnp.bfloat16),
        pltpu.VMEM((RPT, HID), jnp.bfloat16),
        pltpu.VMEM_SHARED((N_PAD, HID), jnp.bfloat16),
        pltpu.VMEM_SHARED((N_PAD, HID), jnp.bfloat16),
        pltpu.SemaphoreType.DMA,
        pltpu.SemaphoreType.DMA,
    ],
)(_agg_body)


def _mm1_body(x_ref, w_ref, d0_ref, d1_ref, g_ref, dinv_ref):
    i = pl.program_id(0)
    deg = 1.0 + d0_ref[...] + d1_ref[...]
    dinv = lax.rsqrt(deg)
    t = jnp.dot(x_ref[...], w_ref[...],
                preferred_element_type=jnp.float32,
                precision=lax.Precision.HIGHEST)
    rows = lax.broadcasted_iota(jnp.int32, (BLK, 1), 0) + i * BLK
    g_ref[...] = jnp.where(rows < N, dinv * t, 0.0).astype(jnp.bfloat16)
    dinv_ref[...] = dinv


def _mm1_call(x, w1, d0, d1):
    return pl.pallas_call(
        _mm1_body,
        grid=(NBLK,),
        in_specs=[
            pl.BlockSpec((BLK, D_IN), lambda i: (i, 0)),
            pl.BlockSpec((D_IN, HID), lambda i: (0, 0)),
            pl.BlockSpec((BLK, 1), lambda i: (i, 0)),
            pl.BlockSpec((BLK, 1), lambda i: (i, 0)),
        ],
        out_specs=[
            pl.BlockSpec((BLK, HID), lambda i: (i, 0)),
            pl.BlockSpec((BLK, 1), lambda i: (i, 0)),
        ],
        out_shape=[
            jax.ShapeDtypeStruct((N_PAD, HID), jnp.bfloat16),
            jax.ShapeDtypeStruct((N_PAD, 1), jnp.float32),
        ],
    )(x, w1, d0, d1)


def _mm2_body(agg_ref, g_ref, dinv_ref, b_ref, w_ref, g2_ref):
    i = pl.program_id(0)
    dinv = dinv_ref[...]
    a = (agg_ref[0].astype(jnp.float32) + agg_ref[1].astype(jnp.float32)
         + g_ref[...].astype(jnp.float32))
    h = jnp.maximum(dinv * a + b_ref[...], 0.0)
    t = jnp.dot(h, w_ref[...],
                preferred_element_type=jnp.float32,
                precision=lax.Precision.HIGHEST)
    rows = lax.broadcasted_iota(jnp.int32, (BLK, 1), 0) + i * BLK
    g2_ref[...] = jnp.where(rows < N, dinv * t, 0.0).astype(jnp.bfloat16)


def _mm2_call(agg, g, dinv, b1r, w2):
    return pl.pallas_call(
        _mm2_body,
        grid=(NBLK,),
        in_specs=[
            pl.BlockSpec((NC, BLK, HID), lambda i: (0, i, 0)),
            pl.BlockSpec((BLK, HID), lambda i: (i, 0)),
            pl.BlockSpec((BLK, 1), lambda i: (i, 0)),
            pl.BlockSpec((1, HID), lambda i: (0, 0)),
            pl.BlockSpec((HID, HID), lambda i: (0, 0)),
        ],
        out_specs=pl.BlockSpec((BLK, HID), lambda i: (i, 0)),
        out_shape=jax.ShapeDtypeStruct((N_PAD, HID), jnp.bfloat16),
    )(agg, g, dinv, b1r, w2)


def _fin_body(agg_ref, g_ref, dinv_ref, b_ref, wfc_ref, bfc_ref, out_ref, acc):
    i = pl.program_id(0)
    dinv = dinv_ref[...]
    a = (agg_ref[0].astype(jnp.float32) + agg_ref[1].astype(jnp.float32)
         + g_ref[...].astype(jnp.float32))
    h = jnp.maximum(dinv * a + b_ref[...], 0.0)
    rows = lax.broadcasted_iota(jnp.int32, (BLK, 1), 0) + i * BLK
    h = jnp.where(rows < N, h, 0.0)
    part = jnp.sum(h, axis=0, keepdims=True)
    acc[...] = jnp.where(i == 0, part, acc[...] + part)

    @pl.when(i == NBLK - 1)
    def _():
        pooled = acc[...] * (1.0 / N)
        out_ref[...] = jnp.dot(pooled, wfc_ref[...],
                               preferred_element_type=jnp.float32,
                               precision=lax.Precision.HIGHEST) + bfc_ref[...]


def _fin_call(agg, g, dinv, b2r, wfcr, bfcr):
    return pl.pallas_call(
        _fin_body,
        grid=(NBLK,),
        in_specs=[
            pl.BlockSpec((NC, BLK, HID), lambda i: (0, i, 0)),
            pl.BlockSpec((BLK, HID), lambda i: (i, 0)),
            pl.BlockSpec((BLK, 1), lambda i: (i, 0)),
            pl.BlockSpec((1, HID), lambda i: (0, 0)),
            pl.BlockSpec((HID, 2), lambda i: (0, 0)),
            pl.BlockSpec((1, 2), lambda i: (0, 0)),
        ],
        out_specs=pl.BlockSpec((1, 2), lambda i: (0, 0)),
        out_shape=jax.ShapeDtypeStruct((1, 2), jnp.float32),
        scratch_shapes=[pltpu.VMEM((1, HID), jnp.float32)],
    )(agg, g, dinv, b2r, wfcr, bfcr)


def kernel(x, edge_index, W1, b1, W2, b2, Wfc, bfc):
    src = edge_index[0]
    dst = edge_index[1]
    pad = jnp.full((E_PAD - E,), N, dtype=jnp.int32)
    src2d = jnp.concatenate([src, pad]).reshape(NW * CPT, CH)
    dst2d = jnp.concatenate([dst, pad]).reshape(NW * CPT, CH)

    deg2 = _deg_call(dst2d)
    d0 = deg2[0][:, None]
    d1 = deg2[1][:, None]

    g1, dinv = _mm1_call(x, W1, d0, d1)
    agg1 = _agg_call(g1, src2d, dst2d)
    g2 = _mm2_call(agg1, g1, dinv, b1.reshape(1, HID), W2)
    agg2 = _agg_call(g2, src2d, dst2d)
    out = _fin_call(agg2, g2, dinv, b2.reshape(1, HID),
                    Wfc, bfc.reshape(1, 2))
    return out.reshape(2)


# speedup vs baseline: 3.1876x; 1.2499x over previous
import functools

import jax
import jax.numpy as jnp
from jax import lax
from jax.experimental import pallas as pl
from jax.experimental.pallas import tpu as pltpu
from jax.experimental.pallas import tpu_sc as plsc

N = 10000
E = 320000
D_IN = 128
HID = 64

NC = 2
NS = 16
NW = NC * NS

CH = 128
TOTAL_CH = E // CH
CH_LO = TOTAL_CH // NW
CH_REM = TOTAL_CH - CH_LO * NW
CPT_MAX = CH_LO + 1
N_PAD = 10240
RPT = N_PAD // NS
BLK = 2560
NBLK = N_PAD // BLK

_mesh = plsc.VectorSubcoreMesh(core_axis_name="c", subcore_axis_name="s")


def _tile_chunks(w):
    nch = jnp.where(w < CH_REM, CH_LO + 1, CH_LO)
    base = w * CH_LO + jnp.minimum(w, CH_REM)
    base_l = jnp.minimum(base, TOTAL_CH - CPT_MAX)
    off = base - base_l
    return base_l, off, nch


def _deg_body(ei_ref, out_ref, dstv, ones_v, zb, deg_sh):
    c = lax.axis_index("c")
    s = lax.axis_index("s")
    w = c * NS + s
    base_l, off, nch = _tile_chunks(w)
    for i in range(8):
        ones_v[pl.ds(i * 16, 16)] = jnp.ones((16,), jnp.float32)
    for i in range(RPT // 16):
        zb[pl.ds(i * 16, 16)] = jnp.zeros((16,), jnp.float32)
    pltpu.sync_copy(zb, deg_sh.at[pl.ds(s * RPT, RPT)])
    pltpu.sync_copy(ei_ref.at[1, pl.ds(base_l, CPT_MAX)], dstv)
    plsc.subcore_barrier()

    def body(j, carry):
        @pl.when(j < nch)
        def _():
            pltpu.sync_copy(ones_v, deg_sh.at[dstv.at[j + off]], add=True)
        return carry

    lax.fori_loop(0, CPT_MAX, body, 0)
    plsc.subcore_barrier()
    pltpu.sync_copy(deg_sh.at[pl.ds(s * RPT, RPT)], zb)
    pltpu.sync_copy(zb, out_ref.at[c, pl.ds(s * RPT, RPT)])


_deg_call = functools.partial(
    pl.kernel,
    out_type=jax.ShapeDtypeStruct((NC, N_PAD), jnp.float32),
    mesh=_mesh,
    compiler_params=pltpu.CompilerParams(use_tc_tiling_on_sc=False),
    scratch_types=[
        pltpu.VMEM((CPT_MAX, CH), jnp.int32),
        pltpu.VMEM((CH,), jnp.float32),
        pltpu.VMEM((RPT,), jnp.float32),
        pltpu.VMEM_SHARED((N_PAD,), jnp.float32),
    ],
)(_deg_body)


def _agg_body(g_ref, ei_ref, out_ref,
              srcv, dstv, rows0, rows1, bounce, g_sh, acc_sh, sem0, sem1):
    c = lax.axis_index("c")
    s = lax.axis_index("s")
    w = c * NS + s
    base_l, off, nch = _tile_chunks(w)

    pltpu.sync_copy(g_ref.at[pl.ds(s * RPT, RPT)], bounce)
    pltpu.sync_copy(bounce, g_sh.at[pl.ds(s * RPT, RPT)])

    def zrow(r, carry):
        for cc in range(HID // 32):
            bounce[r, pl.ds(cc * 32, 32)] = jnp.zeros((32,), jnp.bfloat16)
        return carry

    lax.fori_loop(0, RPT, zrow, 0)
    pltpu.sync_copy(bounce, acc_sh.at[pl.ds(s * RPT, RPT)])
    pltpu.sync_copy(ei_ref.at[0, pl.ds(base_l, CPT_MAX)], srcv)
    pltpu.sync_copy(ei_ref.at[1, pl.ds(base_l, CPT_MAX)], dstv)
    plsc.subcore_barrier()

    def gather(j, rows, sem):
        return pltpu.make_async_copy(g_sh.at[srcv.at[j + off]], rows, sem)

    def scatter_add(j, rows):
        pltpu.sync_copy(rows, acc_sh.at[dstv.at[j + off]], add=True)

    gather(0, rows0, sem0).start()

    def body(i, carry):
        j = 2 * i

        @pl.when(j + 1 < nch)
        def _():
            gather(j + 1, rows1, sem1).start()

        @pl.when(j < nch)
        def _():
            gather(j, rows0, sem0).wait()
            scatter_add(j, rows0)

        @pl.when(j + 2 < nch)
        def _():
            gather(j + 2, rows0, sem0).start()

        @pl.when(j + 1 < nch)
        def _():
            gather(j + 1, rows1, sem1).wait()
            scatter_add(j + 1, rows1)

        return carry

    lax.fori_loop(0, (CPT_MAX + 1) // 2, body, 0)
    plsc.subcore_barrier()
    pltpu.sync_copy(acc_sh.at[pl.ds(s * RPT, RPT)], bounce)
    pltpu.sync_copy(bounce, out_ref.at[c, pl.ds(s * RPT, RPT)])


_agg_call = functools.partial(
    pl.kernel,
    out_type=jax.ShapeDtypeStruct((NC, N_PAD, HID), jnp.bfloat16),
    mesh=_mesh,
    compiler_params=pltpu.CompilerParams(use_tc_tiling_on_sc=False),
    scratch_types=[
        pltpu.VMEM((CPT_MAX, CH), jnp.int32),
        pltpu.VMEM((CPT_MAX, CH), jnp.int32),
        pltpu.VMEM((CH, HID), jnp.bfloat16),
        pltpu.VMEM((CH, HID), jnp.bfloat16),
        pltpu.VMEM((RPT, HID), jnp.bfloat16),
        pltpu.VMEM_SHARED((N_PAD, HID), jnp.bfloat16),
        pltpu.VMEM_SHARED((N_PAD, HID), jnp.bfloat16),
        pltpu.SemaphoreType.DMA,
        pltpu.SemaphoreType.DMA,
    ],
)(_agg_body)


def _mm1_body(x_ref, w_ref, d0_ref, d1_ref, g_ref, dinv_ref):
    i = pl.program_id(0)
    deg = 1.0 + d0_ref[...] + d1_ref[...]
    dinv = lax.rsqrt(deg)
    t = jnp.dot(x_ref[...], w_ref[...],
                preferred_element_type=jnp.float32,
                precision=lax.Precision.HIGHEST)
    rows = lax.broadcasted_iota(jnp.int32, (BLK, 1), 0) + i * BLK
    g_ref[...] = jnp.where(rows < N, dinv * t, 0.0).astype(jnp.bfloat16)
    dinv_ref[...] = dinv


def _mm1_call(x, w1, d0, d1):
    return pl.pallas_call(
        _mm1_body,
        grid=(NBLK,),
        in_specs=[
            pl.BlockSpec((BLK, D_IN), lambda i: (i, 0)),
            pl.BlockSpec((D_IN, HID), lambda i: (0, 0)),
            pl.BlockSpec((BLK, 1), lambda i: (i, 0)),
            pl.BlockSpec((BLK, 1), lambda i: (i, 0)),
        ],
        out_specs=[
            pl.BlockSpec((BLK, HID), lambda i: (i, 0)),
            pl.BlockSpec((BLK, 1), lambda i: (i, 0)),
        ],
        out_shape=[
            jax.ShapeDtypeStruct((N_PAD, HID), jnp.bfloat16),
            jax.ShapeDtypeStruct((N_PAD, 1), jnp.float32),
        ],
    )(x, w1, d0, d1)


def _mm2_body(agg_ref, g_ref, dinv_ref, b_ref, w_ref, g2_ref):
    i = pl.program_id(0)
    dinv = dinv_ref[...]
    a = (agg_ref[0].astype(jnp.float32) + agg_ref[1].astype(jnp.float32)
         + g_ref[...].astype(jnp.float32))
    h = jnp.maximum(dinv * a + b_ref[...], 0.0)
    t = jnp.dot(h, w_ref[...],
                preferred_element_type=jnp.float32,
                precision=lax.Precision.HIGHEST)
    rows = lax.broadcasted_iota(jnp.int32, (BLK, 1), 0) + i * BLK
    g2_ref[...] = jnp.where(rows < N, dinv * t, 0.0).astype(jnp.bfloat16)


def _mm2_call(agg, g, dinv, b1r, w2):
    return pl.pallas_call(
        _mm2_body,
        grid=(NBLK,),
        in_specs=[
            pl.BlockSpec((NC, BLK, HID), lambda i: (0, i, 0)),
            pl.BlockSpec((BLK, HID), lambda i: (i, 0)),
            pl.BlockSpec((BLK, 1), lambda i: (i, 0)),
            pl.BlockSpec((1, HID), lambda i: (0, 0)),
            pl.BlockSpec((HID, HID), lambda i: (0, 0)),
        ],
        out_specs=pl.BlockSpec((BLK, HID), lambda i: (i, 0)),
        out_shape=jax.ShapeDtypeStruct((N_PAD, HID), jnp.bfloat16),
    )(agg, g, dinv, b1r, w2)


def _fin_body(agg_ref, g_ref, dinv_ref, b_ref, wfc_ref, bfc_ref, out_ref, acc):
    i = pl.program_id(0)
    dinv = dinv_ref[...]
    a = (agg_ref[0].astype(jnp.float32) + agg_ref[1].astype(jnp.float32)
         + g_ref[...].astype(jnp.float32))
    h = jnp.maximum(dinv * a + b_ref[...], 0.0)
    rows = lax.broadcasted_iota(jnp.int32, (BLK, 1), 0) + i * BLK
    h = jnp.where(rows < N, h, 0.0)
    part = jnp.sum(h, axis=0, keepdims=True)
    acc[...] = jnp.where(i == 0, part, acc[...] + part)

    @pl.when(i == NBLK - 1)
    def _():
        pooled = acc[...] * (1.0 / N)
        out_ref[...] = jnp.dot(pooled, wfc_ref[...],
                               preferred_element_type=jnp.float32,
                               precision=lax.Precision.HIGHEST) + bfc_ref[...]


def _fin_call(agg, g, dinv, b2r, wfcr, bfcr):
    return pl.pallas_call(
        _fin_body,
        grid=(NBLK,),
        in_specs=[
            pl.BlockSpec((NC, BLK, HID), lambda i: (0, i, 0)),
            pl.BlockSpec((BLK, HID), lambda i: (i, 0)),
            pl.BlockSpec((BLK, 1), lambda i: (i, 0)),
            pl.BlockSpec((1, HID), lambda i: (0, 0)),
            pl.BlockSpec((HID, 2), lambda i: (0, 0)),
            pl.BlockSpec((1, 2), lambda i: (0, 0)),
        ],
        out_specs=pl.BlockSpec((1, 2), lambda i: (0, 0)),
        out_shape=jax.ShapeDtypeStruct((1, 2), jnp.float32),
        scratch_shapes=[pltpu.VMEM((1, HID), jnp.float32)],
    )(agg, g, dinv, b2r, wfcr, bfcr)


def kernel(x, edge_index, W1, b1, W2, b2, Wfc, bfc):
    ei3 = edge_index.reshape(2, TOTAL_CH, CH)

    deg2 = _deg_call(ei3)
    d0 = deg2[0][:, None]
    d1 = deg2[1][:, None]

    g1, dinv = _mm1_call(x, W1, d0, d1)
    agg1 = _agg_call(g1, ei3)
    g2 = _mm2_call(agg1, g1, dinv, b1.reshape(1, HID), W2)
    agg2 = _agg_call(g2, ei3)
    out = _fin_call(agg2, g2, dinv, b2.reshape(1, HID),
                    Wfc, bfc.reshape(1, 2))
    return out.reshape(2)


# speedup vs baseline: 3.3433x; 1.0488x over previous
import functools

import jax
import jax.numpy as jnp
from jax import lax
from jax.experimental import pallas as pl
from jax.experimental.pallas import tpu as pltpu
from jax.experimental.pallas import tpu_sc as plsc

N = 10000
E = 320000
D_IN = 128
HID = 64

NC = 2
NS = 16
NW = NC * NS

CH = 128
TOTAL_CH = E // CH
CH_LO = TOTAL_CH // NW
CH_REM = TOTAL_CH - CH_LO * NW
CPT_MAX = CH_LO + 1
N_PAD = 10240
RPT = N_PAD // NS
BLK = 2560
NBLK = N_PAD // BLK

_mesh = plsc.VectorSubcoreMesh(core_axis_name="c", subcore_axis_name="s")


def _tile_chunks(w):
    nch = jnp.where(w < CH_REM, CH_LO + 1, CH_LO)
    base = w * CH_LO + jnp.minimum(w, CH_REM)
    base_l = jnp.minimum(base, TOTAL_CH - CPT_MAX)
    off = base - base_l
    return base_l, off, nch


def _deg_body(ei_ref, out_ref, dstv, ones_v, zb, deg_sh, dsem0, dsem1):
    c = lax.axis_index("c")
    s = lax.axis_index("s")
    w = c * NS + s
    base_l, off, nch = _tile_chunks(w)
    for i in range(8):
        ones_v[pl.ds(i * 16, 16)] = jnp.ones((16,), jnp.float32)
    for i in range(RPT // 16):
        zb[pl.ds(i * 16, 16)] = jnp.zeros((16,), jnp.float32)
    pltpu.sync_copy(zb, deg_sh.at[pl.ds(s * RPT, RPT)])
    pltpu.sync_copy(ei_ref.at[1, pl.ds(base_l, CPT_MAX)], dstv)
    plsc.subcore_barrier()

    def ones_wait(sem):
        return pltpu.make_async_copy(ones_v, deg_sh.at[dstv.at[off]], sem)

    def body(i, carry):
        for par, sem in ((0, dsem0), (1, dsem1)):
            j = 2 * i + par

            @pl.when(j < nch)
            def _():
                @pl.when(j >= 2)
                def _():
                    ones_wait(sem).wait()
                pltpu.async_copy(ones_v, deg_sh.at[dstv.at[j + off]], sem,
                                 add=True)

        return carry

    lax.fori_loop(0, (CPT_MAX + 1) // 2, body, 0)
    ones_wait(dsem0).wait()
    ones_wait(dsem1).wait()
    plsc.subcore_barrier()
    pltpu.sync_copy(deg_sh.at[pl.ds(s * RPT, RPT)], zb)
    pltpu.sync_copy(zb, out_ref.at[c, pl.ds(s * RPT, RPT)])


_deg_call = functools.partial(
    pl.kernel,
    out_type=jax.ShapeDtypeStruct((NC, N_PAD), jnp.float32),
    mesh=_mesh,
    compiler_params=pltpu.CompilerParams(use_tc_tiling_on_sc=False),
    scratch_types=[
        pltpu.VMEM((CPT_MAX, CH), jnp.int32),
        pltpu.VMEM((CH,), jnp.float32),
        pltpu.VMEM((RPT,), jnp.float32),
        pltpu.VMEM_SHARED((N_PAD,), jnp.float32),
        pltpu.SemaphoreType.DMA,
        pltpu.SemaphoreType.DMA,
    ],
)(_deg_body)


def _agg_body(g_ref, ei_ref, out_ref,
              srcv, dstv, rows0, rows1, bounce, g_sh, acc_sh, sem0, sem1):
    c = lax.axis_index("c")
    s = lax.axis_index("s")
    w = c * NS + s
    base_l, off, nch = _tile_chunks(w)

    pltpu.sync_copy(g_ref.at[pl.ds(s * RPT, RPT)], bounce)
    pltpu.sync_copy(bounce, g_sh.at[pl.ds(s * RPT, RPT)])
    pltpu.sync_copy(bounce, acc_sh.at[pl.ds(s * RPT, RPT)])
    pltpu.sync_copy(ei_ref.at[0, pl.ds(base_l, CPT_MAX)], srcv)
    pltpu.sync_copy(ei_ref.at[1, pl.ds(base_l, CPT_MAX)], dstv)
    plsc.subcore_barrier()

    def gather(j, rows, sem):
        return pltpu.make_async_copy(g_sh.at[srcv.at[j + off]], rows, sem)

    def scatter_add(j, rows):
        pltpu.sync_copy(rows, acc_sh.at[dstv.at[j + off]], add=True)

    gather(0, rows0, sem0).start()

    def body(i, carry):
        j = 2 * i

        @pl.when(j + 1 < nch)
        def _():
            gather(j + 1, rows1, sem1).start()

        @pl.when(j < nch)
        def _():
            gather(j, rows0, sem0).wait()
            scatter_add(j, rows0)

        @pl.when(j + 2 < nch)
        def _():
            gather(j + 2, rows0, sem0).start()

        @pl.when(j + 1 < nch)
        def _():
            gather(j + 1, rows1, sem1).wait()
            scatter_add(j + 1, rows1)

        return carry

    lax.fori_loop(0, (CPT_MAX + 1) // 2, body, 0)
    plsc.subcore_barrier()
    pltpu.sync_copy(acc_sh.at[pl.ds(s * RPT, RPT)], bounce)
    pltpu.sync_copy(bounce, out_ref.at[c, pl.ds(s * RPT, RPT)])


_agg_call = functools.partial(
    pl.kernel,
    out_type=jax.ShapeDtypeStruct((NC, N_PAD, HID), jnp.bfloat16),
    mesh=_mesh,
    compiler_params=pltpu.CompilerParams(use_tc_tiling_on_sc=False),
    scratch_types=[
        pltpu.VMEM((CPT_MAX, CH), jnp.int32),
        pltpu.VMEM((CPT_MAX, CH), jnp.int32),
        pltpu.VMEM((CH, HID), jnp.bfloat16),
        pltpu.VMEM((CH, HID), jnp.bfloat16),
        pltpu.VMEM((RPT, HID), jnp.bfloat16),
        pltpu.VMEM_SHARED((N_PAD, HID), jnp.bfloat16),
        pltpu.VMEM_SHARED((N_PAD, HID), jnp.bfloat16),
        pltpu.SemaphoreType.DMA,
        pltpu.SemaphoreType.DMA,
    ],
)(_agg_body)


def _mm1_body(x_ref, w_ref, d0_ref, d1_ref, g_ref, dinv_ref):
    i = pl.program_id(0)
    deg = 1.0 + d0_ref[...] + d1_ref[...]
    dinv = lax.rsqrt(deg)
    t = jnp.dot(x_ref[...], w_ref[...],
                preferred_element_type=jnp.float32,
                precision=lax.Precision.HIGHEST)
    rows = lax.broadcasted_iota(jnp.int32, (BLK, 1), 0) + i * BLK
    g_ref[...] = jnp.where(rows < N, dinv * t, 0.0).astype(jnp.bfloat16)
    dinv_ref[...] = dinv


def _mm1_call(x, w1, d0, d1):
    return pl.pallas_call(
        _mm1_body,
        grid=(NBLK,),
        in_specs=[
            pl.BlockSpec((BLK, D_IN), lambda i: (i, 0)),
            pl.BlockSpec((D_IN, HID), lambda i: (0, 0)),
            pl.BlockSpec((BLK, 1), lambda i: (i, 0)),
            pl.BlockSpec((BLK, 1), lambda i: (i, 0)),
        ],
        out_specs=[
            pl.BlockSpec((BLK, HID), lambda i: (i, 0)),
            pl.BlockSpec((BLK, 1), lambda i: (i, 0)),
        ],
        out_shape=[
            jax.ShapeDtypeStruct((N_PAD, HID), jnp.bfloat16),
            jax.ShapeDtypeStruct((N_PAD, 1), jnp.float32),
        ],
    )(x, w1, d0, d1)


def _mm2_body(agg_ref, g_ref, dinv_ref, b_ref, w_ref, g2_ref):
    i = pl.program_id(0)
    dinv = dinv_ref[...]
    a = (agg_ref[0].astype(jnp.float32) + agg_ref[1].astype(jnp.float32)
         - g_ref[...].astype(jnp.float32))
    h = jnp.maximum(dinv * a + b_ref[...], 0.0)
    t = jnp.dot(h, w_ref[...],
                preferred_element_type=jnp.float32,
                precision=lax.Precision.HIGHEST)
    rows = lax.broadcasted_iota(jnp.int32, (BLK, 1), 0) + i * BLK
    g2_ref[...] = jnp.where(rows < N, dinv * t, 0.0).astype(jnp.bfloat16)


def _mm2_call(agg, g, dinv, b1r, w2):
    return pl.pallas_call(
        _mm2_body,
        grid=(NBLK,),
        in_specs=[
            pl.BlockSpec((NC, BLK, HID), lambda i: (0, i, 0)),
            pl.BlockSpec((BLK, HID), lambda i: (i, 0)),
            pl.BlockSpec((BLK, 1), lambda i: (i, 0)),
            pl.BlockSpec((1, HID), lambda i: (0, 0)),
            pl.BlockSpec((HID, HID), lambda i: (0, 0)),
        ],
        out_specs=pl.BlockSpec((BLK, HID), lambda i: (i, 0)),
        out_shape=jax.ShapeDtypeStruct((N_PAD, HID), jnp.bfloat16),
    )(agg, g, dinv, b1r, w2)


def _fin_body(agg_ref, g_ref, dinv_ref, b_ref, wfc_ref, bfc_ref, out_ref, acc):
    i = pl.program_id(0)
    dinv = dinv_ref[...]
    a = (agg_ref[0].astype(jnp.float32) + agg_ref[1].astype(jnp.float32)
         - g_ref[...].astype(jnp.float32))
    h = jnp.maximum(dinv * a + b_ref[...], 0.0)
    rows = lax.broadcasted_iota(jnp.int32, (BLK, 1), 0) + i * BLK
    h = jnp.where(rows < N, h, 0.0)
    part = jnp.sum(h, axis=0, keepdims=True)
    acc[...] = jnp.where(i == 0, part, acc[...] + part)

    @pl.when(i == NBLK - 1)
    def _():
        pooled = acc[...] * (1.0 / N)
        out_ref[...] = jnp.dot(pooled, wfc_ref[...],
                               preferred_element_type=jnp.float32,
                               precision=lax.Precision.HIGHEST) + bfc_ref[...]


def _fin_call(agg, g, dinv, b2r, wfcr, bfcr):
    return pl.pallas_call(
        _fin_body,
        grid=(NBLK,),
        in_specs=[
            pl.BlockSpec((NC, BLK, HID), lambda i: (0, i, 0)),
            pl.BlockSpec((BLK, HID), lambda i: (i, 0)),
            pl.BlockSpec((BLK, 1), lambda i: (i, 0)),
            pl.BlockSpec((1, HID), lambda i: (0, 0)),
            pl.BlockSpec((HID, 2), lambda i: (0, 0)),
            pl.BlockSpec((1, 2), lambda i: (0, 0)),
        ],
        out_specs=pl.BlockSpec((1, 2), lambda i: (0, 0)),
        out_shape=jax.ShapeDtypeStruct((1, 2), jnp.float32),
        scratch_shapes=[pltpu.VMEM((1, HID), jnp.float32)],
    )(agg, g, dinv, b2r, wfcr, bfcr)


def kernel(x, edge_index, W1, b1, W2, b2, Wfc, bfc):
    ei3 = edge_index.reshape(2, TOTAL_CH, CH)

    deg2 = _deg_call(ei3)
    d0 = deg2[0][:, None]
    d1 = deg2[1][:, None]

    g1, dinv = _mm1_call(x, W1, d0, d1)
    agg1 = _agg_call(g1, ei3)
    g2 = _mm2_call(agg1, g1, dinv, b1.reshape(1, HID), W2)
    agg2 = _agg_call(g2, ei3)
    out = _fin_call(agg2, g2, dinv, b2.reshape(1, HID),
                    Wfc, bfc.reshape(1, 2))
    return out.reshape(2)
